# baseline (speedup 1.0000x reference)
import functools

import jax
import jax.numpy as jnp
from jax import lax
from jax.experimental import pallas as pl
from jax.experimental.pallas import tpu as pltpu
from jax.experimental.pallas import tpu_sc as plsc

_NUM_CORES = 2
_NUM_SUBCORES = 16
_NW = _NUM_CORES * _NUM_SUBCORES
_L = 16

_LN2 = 0.6931471805599453
_SQRT2 = 1.4142135623730951


def _vln(s):
    bits = lax.bitcast_convert_type(s, jnp.int32)
    e = (bits >> 23) - 127
    m = lax.bitcast_convert_type(
        (bits & jnp.int32(0x007FFFFF)) | jnp.int32(0x3F800000), jnp.float32)
    big = m > _SQRT2
    m = jnp.where(big, m * 0.5, m)
    e = (e + big.astype(jnp.int32)).astype(jnp.float32)
    r = (m - 1.0) / (m + 1.0)
    r2 = r * r
    p = 2.0 * r * (1.0 + r2 * (1.0 / 3.0 + r2 * (0.2 + r2 * (1.0 / 7.0 + r2 * (1.0 / 9.0)))))
    return e * _LN2 + p


@functools.lru_cache(maxsize=None)
def _make_sc_logsoftmax_gather(N, B):
    bpw = B // _NW
    groups = bpw // _L
    mesh = plsc.VectorSubcoreMesh(core_axis_name="c", subcore_axis_name="s")

    @functools.partial(
        pl.kernel,
        mesh=mesh,
        out_type=jax.ShapeDtypeStruct((B, N), jnp.float32),
        scratch_types=[
            pltpu.VMEM((bpw,), jnp.int32),
            pltpu.VMEM((bpw, N), jnp.float32),
            pltpu.SemaphoreType.DMA,
        ],
        compiler_params=pltpu.CompilerParams(needs_layout_passes=False),
    )
    def fused(table_hbm, idx_hbm, out_hbm, idx_v, rows_v, sem):
        wid = lax.axis_index("s") * _NUM_CORES + lax.axis_index("c")
        base = wid * bpw
        pltpu.sync_copy(idx_hbm.at[pl.ds(base, bpw)], idx_v)
        pltpu.async_copy(table_hbm.at[idx_v], rows_v, sem).wait()

        lane = lax.iota(jnp.int32, _L)
        one = jnp.ones((_L,), jnp.int32)

        def group_body(g, carry):
            row = g * _L + lane
            col = jnp.zeros((_L,), jnp.int32)
            m = plsc.load_gather(rows_v, [row, col])
            for _ in range(1, N):
                col = col + one
                m = jnp.maximum(m, plsc.load_gather(rows_v, [row, col]))
            col = jnp.zeros((_L,), jnp.int32)
            s = jnp.exp(plsc.load_gather(rows_v, [row, col]) - m)
            for _ in range(1, N):
                col = col + one
                s = s + jnp.exp(plsc.load_gather(rows_v, [row, col]) - m)
            negc = -(m + _vln(s))
            col = jnp.zeros((_L,), jnp.int32)
            plsc.addupdate_scatter(rows_v, [row, col], negc)
            for _ in range(1, N):
                col = col + one
                plsc.addupdate_scatter(rows_v, [row, col], negc)
            return carry

        lax.fori_loop(0, groups, group_body, 0)
        pltpu.sync_copy(rows_v, out_hbm.at[pl.ds(base, bpw)])

    return fused


def kernel(M, z):
    _, N = M.shape
    B = z.shape[0]
    return _make_sc_logsoftmax_gather(N, B)(M, z.astype(jnp.int32))


# speedup vs baseline: 1.0625x; 1.0625x over previous
import functools

import jax
import jax.numpy as jnp
from jax import lax
from jax.experimental import pallas as pl
from jax.experimental.pallas import tpu as pltpu
from jax.experimental.pallas import tpu_sc as plsc

_NUM_CORES = 2
_NUM_SUBCORES = 16
_NW = _NUM_CORES * _NUM_SUBCORES
_L = 16

_LN2 = 0.6931471805599453
_SQRT2 = 1.4142135623730951


def _vln(s):
    bits = lax.bitcast_convert_type(s, jnp.int32)
    e = (bits >> 23) - 127
    m = lax.bitcast_convert_type(
        (bits & jnp.int32(0x007FFFFF)) | jnp.int32(0x3F800000), jnp.float32)
    big = m > _SQRT2
    m = jnp.where(big, m * 0.5, m)
    e = (e + big.astype(jnp.int32)).astype(jnp.float32)
    r = (m - 1.0) / (m + 1.0)
    r2 = r * r
    p = 2.0 * r * (1.0 + r2 * (1.0 / 3.0 + r2 * (0.2 + r2 * (1.0 / 7.0 + r2 * (1.0 / 9.0)))))
    return e * _LN2 + p


@functools.lru_cache(maxsize=None)
def _make_sc_logsoftmax_gather(N, B):
    bpw = B // _NW
    groups = bpw // _L
    mesh = plsc.VectorSubcoreMesh(core_axis_name="c", subcore_axis_name="s")

    @functools.partial(
        pl.kernel,
        mesh=mesh,
        out_type=jax.ShapeDtypeStruct((B, N), jnp.float32),
        scratch_types=[
            pltpu.VMEM((bpw,), jnp.int32),
            pltpu.VMEM((bpw, N), jnp.float32),
            pltpu.SemaphoreType.DMA,
        ],
        compiler_params=pltpu.CompilerParams(needs_layout_passes=False),
    )
    def fused(table_hbm, idx_hbm, out_hbm, idx_v, rows_v, sem):
        wid = lax.axis_index("s") * _NUM_CORES + lax.axis_index("c")
        base = wid * bpw
        pltpu.sync_copy(idx_hbm.at[pl.ds(base, bpw)], idx_v)
        pltpu.async_copy(table_hbm.at[idx_v], rows_v, sem).wait()

        lane = lax.iota(jnp.int32, _L)
        cols = [jnp.full((_L,), j, jnp.int32) for j in range(N)]
        nacc = 8

        def _tree(vals, op):
            while len(vals) > 1:
                vals = [op(vals[i], vals[i + 1]) for i in range(0, len(vals), 2)]
            return vals[0]

        def group_body(g, carry):
            row = g * _L + lane
            macc = [plsc.load_gather(rows_v, [row, cols[a]]) for a in range(nacc)]
            for j in range(nacc, N):
                macc[j % nacc] = jnp.maximum(
                    macc[j % nacc], plsc.load_gather(rows_v, [row, cols[j]]))
            m = _tree(macc, jnp.maximum)
            sacc = [jnp.exp(plsc.load_gather(rows_v, [row, cols[a]]) - m)
                    for a in range(nacc)]
            for j in range(nacc, N):
                sacc[j % nacc] = sacc[j % nacc] + jnp.exp(
                    plsc.load_gather(rows_v, [row, cols[j]]) - m)
            s = _tree(sacc, jnp.add)
            negc = -(m + _vln(s))
            for j in range(N):
                plsc.addupdate_scatter(rows_v, [row, cols[j]], negc)
            return carry

        lax.fori_loop(0, groups, group_body, 0)
        pltpu.sync_copy(rows_v, out_hbm.at[pl.ds(base, bpw)])

    return fused


def kernel(M, z):
    _, N = M.shape
    B = z.shape[0]
    return _make_sc_logsoftmax_gather(N, B)(M, z.astype(jnp.int32))


# speedup vs baseline: 3.4891x; 3.2839x over previous
import functools

import jax
import jax.numpy as jnp
from jax import lax
from jax.experimental import pallas as pl
from jax.experimental.pallas import tpu as pltpu
from jax.experimental.pallas import tpu_sc as plsc

_NUM_CORES = 2
_NUM_SUBCORES = 16
_NW = _NUM_CORES * _NUM_SUBCORES
_L = 16
_R = 8

_LN2 = 0.6931471805599453
_FOLD = 4.0 / 3.0


def _ln_scalar(s):
    bits = lax.bitcast_convert_type(s, jnp.int32)
    e = (bits >> 23) - 127
    m = lax.bitcast_convert_type(
        (bits & jnp.int32(0x007FFFFF)) | jnp.int32(0x3F800000), jnp.float32)
    big = m >= _FOLD
    m = jnp.where(big, m * 0.5, m)
    e = (e + big.astype(jnp.int32)).astype(jnp.float32)
    t = m - 1.0
    p = jnp.float32(1.0 / 9.0)
    for k in (8, 7, 6, 5, 4, 3, 2):
        p = p * t - jnp.float32((-1.0) ** k / k)
    p = (p * t + 1.0) * t
    return e * _LN2 + p


def _tree(vals, op):
    while len(vals) > 1:
        vals = [op(vals[i], vals[i + 1]) for i in range(0, len(vals), 2)]
    return vals[0]


@functools.lru_cache(maxsize=None)
def _make_sc_logsoftmax_gather(N, B):
    bpw = B // _NW
    nv = N // _L
    mesh = plsc.VectorSubcoreMesh(core_axis_name="c", subcore_axis_name="s")

    @functools.partial(
        pl.kernel,
        mesh=mesh,
        out_type=jax.ShapeDtypeStruct((B, N), jnp.float32),
        scratch_types=[
            pltpu.VMEM((bpw,), jnp.int32),
            pltpu.VMEM((bpw, N), jnp.float32),
            pltpu.SemaphoreType.DMA,
        ],
        compiler_params=pltpu.CompilerParams(needs_layout_passes=False),
    )
    def fused(table_hbm, idx_hbm, out_hbm, idx_v, rows_v, sem):
        wid = lax.axis_index("s") * _NUM_CORES + lax.axis_index("c")
        base = wid * bpw
        pltpu.sync_copy(idx_hbm.at[pl.ds(base, bpw)], idx_v)
        pltpu.async_copy(table_hbm.at[idx_v], rows_v, sem).wait()

        def row_logsoftmax(r):
            x = [rows_v[r, pl.ds(j * _L, _L)] for j in range(nv)]
            m = jnp.max(_tree(x, jnp.maximum))
            s = jnp.sum(_tree([jnp.exp(xj - m) for xj in x], jnp.add))
            negc = jnp.full((_L,), -(m + _ln_scalar(s)), jnp.float32)
            for j in range(nv):
                plsc.addupdate(rows_v.at[r, pl.ds(j * _L, _L)], negc)

        def block_body(i, carry):
            r0 = i * _R
            for u in range(_R):
                row_logsoftmax(r0 + u)
            return carry

        lax.fori_loop(0, bpw // _R, block_body, 0)
        pltpu.sync_copy(rows_v, out_hbm.at[pl.ds(base, bpw)])

    return fused


def kernel(M, z):
    _, N = M.shape
    B = z.shape[0]
    return _make_sc_logsoftmax_gather(N, B)(M, z.astype(jnp.int32))


# speedup vs baseline: 3.6841x; 1.0559x over previous
import functools

import jax
import jax.numpy as jnp
from jax import lax
from jax.experimental import pallas as pl
from jax.experimental.pallas import tpu as pltpu
from jax.experimental.pallas import tpu_sc as plsc

_NUM_CORES = 2
_NUM_SUBCORES = 16
_NW = _NUM_CORES * _NUM_SUBCORES
_L = 16
_R = 8

_LN2 = 0.6931471805599453
_FOLD = 4.0 / 3.0


def _ln_scalar(s):
    bits = lax.bitcast_convert_type(s, jnp.int32)
    e = (bits >> 23) - 127
    m = lax.bitcast_convert_type(
        (bits & jnp.int32(0x007FFFFF)) | jnp.int32(0x3F800000), jnp.float32)
    big = m >= _FOLD
    m = jnp.where(big, m * 0.5, m)
    e = (e + big.astype(jnp.int32)).astype(jnp.float32)
    t = m - 1.0
    p = jnp.float32(1.0 / 9.0)
    for k in (8, 7, 6, 5, 4, 3, 2):
        p = p * t - jnp.float32((-1.0) ** k / k)
    p = (p * t + 1.0) * t
    return e * _LN2 + p


def _tree(vals, op):
    while len(vals) > 1:
        vals = [op(vals[i], vals[i + 1]) for i in range(0, len(vals), 2)]
    return vals[0]


_NCHUNK = 4


@functools.lru_cache(maxsize=None)
def _make_sc_logsoftmax_gather(N, B):
    bpw = B // _NW
    nv = N // _L
    crows = bpw // _NCHUNK
    mesh = plsc.VectorSubcoreMesh(core_axis_name="c", subcore_axis_name="s")

    @functools.partial(
        pl.kernel,
        mesh=mesh,
        out_type=jax.ShapeDtypeStruct((B, N), jnp.float32),
        scratch_types=[
            pltpu.VMEM((bpw,), jnp.int32),
            pltpu.VMEM((bpw, N), jnp.float32),
            pltpu.SemaphoreType.DMA,
            pltpu.SemaphoreType.DMA,
        ],
        compiler_params=pltpu.CompilerParams(needs_layout_passes=False),
    )
    def fused(table_hbm, idx_hbm, out_hbm, idx_v, rows_v, in_sem, out_sem):
        wid = lax.axis_index("s") * _NUM_CORES + lax.axis_index("c")
        base = wid * bpw
        pltpu.sync_copy(idx_hbm.at[pl.ds(base, bpw)], idx_v)
        gathers = [
            pltpu.async_copy(
                table_hbm.at[idx_v.at[pl.ds(c * crows, crows)]],
                rows_v.at[pl.ds(c * crows, crows)],
                in_sem,
            )
            for c in range(_NCHUNK)
        ]

        def row_logsoftmax(r):
            x = [rows_v[r, pl.ds(j * _L, _L)] for j in range(nv)]
            m = jnp.max(_tree(x, jnp.maximum))
            s = jnp.sum(_tree([jnp.exp(xj - m) for xj in x], jnp.add))
            negc = jnp.full((_L,), -(m + _ln_scalar(s)), jnp.float32)
            for j in range(nv):
                plsc.addupdate(rows_v.at[r, pl.ds(j * _L, _L)], negc)

        writebacks = []
        for c in range(_NCHUNK):
            gathers[c].wait()

            def block_body(i, carry, c=c):
                r0 = c * crows + i * _R
                for u in range(_R):
                    row_logsoftmax(r0 + u)
                return carry

            lax.fori_loop(0, crows // _R, block_body, 0)
            writebacks.append(
                pltpu.async_copy(
                    rows_v.at[pl.ds(c * crows, crows)],
                    out_hbm.at[pl.ds(base + c * crows, crows)],
                    out_sem,
                )
            )
        for wb in writebacks:
            wb.wait()

    return fused


def kernel(M, z):
    _, N = M.shape
    B = z.shape[0]
    return _make_sc_logsoftmax_gather(N, B)(M, z.astype(jnp.int32))
